# baseline (speedup 1.0000x reference)
import functools
import jax
import jax.numpy as jnp
from jax import lax
from jax.experimental import pallas as pl
from jax.experimental.pallas import tpu as pltpu
from jax.experimental.pallas import tpu_sc as plsc

N = 10000
E = 320000
D = 128

NW = 32
RPW = 320
NPAD = NW * RPW
CHUNK = 8000
NCHUNKS = E // CHUNK
VECS = CHUNK // 16
GROUP = 10
NGROUPS = VECS // GROUP
GB = 128
SLACK = GROUP * 16
SELC = GB + SLACK
DSTBITS = 9


def _sc_agg(fts, u_arr, v_arr):
    mesh = plsc.VectorSubcoreMesh(core_axis_name="c", subcore_axis_name="s")

    @functools.partial(
        pl.kernel,
        mesh=mesh,
        out_type=jax.ShapeDtypeStruct((NPAD, D), jnp.float32),
        scratch_types=[
            pltpu.VMEM((RPW + 1, D), jnp.float32),
            pltpu.VMEM((CHUNK,), jnp.int32),
            pltpu.VMEM((CHUNK,), jnp.int32),
            pltpu.VMEM((CHUNK,), jnp.int32),
            pltpu.VMEM((CHUNK,), jnp.int32),
            pltpu.VMEM((SELC,), jnp.int32),
            pltpu.VMEM((GB,), jnp.int32),
            pltpu.VMEM((GB,), jnp.int32),
            pltpu.VMEM((GB, D), jnp.float32),
            pltpu.VMEM((GB,), jnp.int32),
            pltpu.VMEM((GB,), jnp.int32),
            pltpu.VMEM((GB, D), jnp.float32),
            pltpu.SemaphoreType.DMA,
            pltpu.SemaphoreType.DMA,
            pltpu.SemaphoreType.DMA,
        ],
        compiler_params=pltpu.CompilerParams(needs_layout_passes=False),
    )
    def k(fts_hbm, u_hbm, v_hbm, out_hbm, acc, ub0, ub1, vb0, vb1, selc,
          selvA, seldA, rowsA, selvB, seldB, rowsB, esem, gsemA, gsemB):
        wid = lax.axis_index("s") * 2 + lax.axis_index("c")
        lo = wid * RPW

        zero16 = jnp.zeros((16,), jnp.float32)

        def zbody(i, _):
            for j in range(D // 16):
                acc[i, pl.ds(j * 16, 16)] = zero16
            return 0

        lax.fori_loop(0, RPW + 1, zbody, 0)

        def start_chunk(c, ubuf, vbuf):
            pltpu.async_copy(u_hbm.at[pl.ds(c * CHUNK, CHUNK)], ubuf, esem)
            pltpu.async_copy(v_hbm.at[pl.ds(c * CHUNK, CHUNK)], vbuf, esem)

        def wait_chunk(c, ubuf, vbuf):
            pltpu.make_async_copy(u_hbm.at[pl.ds(c * CHUNK, CHUNK)], ubuf, esem).wait()
            pltpu.make_async_copy(v_hbm.at[pl.ds(c * CHUNK, CHUNK)], vbuf, esem).wait()

        def decode_issue(selv, seld, rows, gsem):
            def dbody(j, _):
                sl = pl.ds(j * 16, 16)
                comb = selc[sl]
                selv[sl] = comb >> DSTBITS
                seld[sl] = comb & ((1 << DSTBITS) - 1)
                return 0

            lax.fori_loop(0, GB // 16, dbody, 0)
            pltpu.async_copy(fts_hbm.at[selv], rows, gsem)

        def wait_accum(selv, seld, rows, gsem):
            pltpu.make_async_copy(fts_hbm.at[selv], rows, gsem).wait()

            def abody(g, _):
                dstv = seld[pl.ds(g * 16, 16)]
                for t in range(16):
                    dst = dstv[t]
                    i = g * 16 + t
                    for j in range(D // 16):
                        sl = pl.ds(j * 16, 16)
                        acc[dst, sl] = jnp.maximum(acc[dst, sl], rows[i, sl])
                return 0

            lax.fori_loop(0, GB // 16, abody, 0)

        def fire_cond(state):
            cnt_vec, par, pend = state
            return cnt_vec[0] >= GB

        def fire_body(state):
            cnt_vec, par, pend = state

            @pl.when(par == 0)
            def _():
                decode_issue(selvA, seldA, rowsA, gsemA)

                @pl.when(pend == 1)
                def _():
                    wait_accum(selvB, seldB, rowsB, gsemB)

            @pl.when(par == 1)
            def _():
                decode_issue(selvB, seldB, rowsB, gsemB)

                @pl.when(pend == 1)
                def _():
                    wait_accum(selvA, seldA, rowsA, gsemA)

            for i in range(SLACK // 16):
                selc[pl.ds(i * 16, 16)] = selc[pl.ds(GB + i * 16, 16)]

            return (cnt_vec - GB, 1 - par, jnp.int32(1))

        def scan_chunk(ubuf, vbuf, state):
            def group_body(g, state):
                cnt_vec, par, pend = state
                base = g * (GROUP * 16)
                for t in range(GROUP):
                    sl = pl.ds(base + t * 16, 16)
                    uv = ubuf[sl]
                    vv = vbuf[sl]
                    rel = uv - lo
                    msk = (rel >= 0) & (rel < RPW)
                    pos = plsc.cumsum(msk.astype(jnp.int32))
                    idx = cnt_vec + pos - 1
                    comb = (vv << DSTBITS) | rel
                    plsc.store_scatter(selc, [idx], comb, mask=msk)
                    cnt_vec = cnt_vec + plsc.all_reduce_population_count(msk)
                return lax.while_loop(fire_cond, fire_body, (cnt_vec, par, pend))

            return lax.fori_loop(0, NGROUPS, group_body, state)

        start_chunk(0, ub0, vb0)
        state = (jnp.zeros((16,), jnp.int32), jnp.int32(0), jnp.int32(0))

        def chunk_body(c2, state):
            a = 2 * c2
            start_chunk(a + 1, ub1, vb1)
            wait_chunk(a, ub0, vb0)
            state = scan_chunk(ub0, vb0, state)

            @pl.when(a + 2 < NCHUNKS)
            def _():
                start_chunk(a + 2, ub0, vb0)

            wait_chunk(a + 1, ub1, vb1)
            state = scan_chunk(ub1, vb1, state)
            return state

        cnt_vec, par, pend = lax.fori_loop(0, NCHUNKS // 2, chunk_body, state)

        lane = lax.iota(jnp.int32, 16)

        def pbody(i, _):
            sl = pl.ds(i * 16, 16)
            live = lane + (i * 16) < cnt_vec
            selc[sl] = jnp.where(live, selc[sl], RPW)
            return 0

        lax.fori_loop(0, SELC // 16, pbody, 0)

        tail_cnt = jnp.where(cnt_vec[0] > 0,
                             jnp.full((16,), GB, jnp.int32),
                             jnp.zeros((16,), jnp.int32))
        cnt_vec, par, pend = lax.while_loop(
            fire_cond, fire_body, (tail_cnt, par, pend))

        @pl.when((pend == 1) & (par == 1))
        def _():
            wait_accum(selvA, seldA, rowsA, gsemA)

        @pl.when((pend == 1) & (par == 0))
        def _():
            wait_accum(selvB, seldB, rowsB, gsemB)

        pltpu.sync_copy(acc.at[pl.ds(0, RPW)], out_hbm.at[pl.ds(lo, RPW)])

    return k(fts, u_arr, v_arr)


def _tc_body(fts_ref, agg_ref, w1_ref, w2_ref, out_ref):
    y = jnp.dot(fts_ref[...], w1_ref[...], preferred_element_type=jnp.float32)
    y = y + jnp.dot(agg_ref[...], w2_ref[...], preferred_element_type=jnp.float32)
    nrm = jnp.sum(y * y, axis=1, keepdims=True)
    out_ref[...] = y * lax.rsqrt(nrm)


def _tc_linear(fts, agg, W_l):
    w1 = W_l[:, :D].T
    w2 = W_l[:, D:].T
    B = 1000
    grid = (N // B,)
    return pl.pallas_call(
        _tc_body,
        grid=grid,
        in_specs=[
            pl.BlockSpec((B, D), lambda i: (i, 0)),
            pl.BlockSpec((B, D), lambda i: (i, 0)),
            pl.BlockSpec((D, D), lambda i: (0, 0)),
            pl.BlockSpec((D, D), lambda i: (0, 0)),
        ],
        out_specs=pl.BlockSpec((B, D), lambda i: (i, 0)),
        out_shape=jax.ShapeDtypeStruct((N, D), jnp.float32),
    )(fts, agg, w1, w2)


def kernel(fts, edge_index, W_l):
    agg = _sc_agg(fts, edge_index[0], edge_index[1])[:N]
    return _tc_linear(fts, agg, W_l)


# speedup vs baseline: 1.3954x; 1.3954x over previous
import functools
import jax
import jax.numpy as jnp
from jax import lax
from jax.experimental import pallas as pl
from jax.experimental.pallas import tpu as pltpu
from jax.experimental.pallas import tpu_sc as plsc

N = 10000
E = 320000
D = 128

NW = 32
RPW = 320
NPAD = NW * RPW
CHUNK = 8000
NCHUNKS = E // CHUNK
VECS = CHUNK // 16
GROUP = 10
NGROUPS = VECS // GROUP
GB = 128
SLACK = GROUP * 16
SELC = GB + SLACK
DSTBITS = 9


def _sc_agg(fts, u_arr, v_arr):
    mesh = plsc.VectorSubcoreMesh(core_axis_name="c", subcore_axis_name="s")

    @functools.partial(
        pl.kernel,
        mesh=mesh,
        out_type=jax.ShapeDtypeStruct((NPAD, D), jnp.float32),
        scratch_types=[
            pltpu.VMEM((RPW + 1, D), jnp.float32),
            pltpu.VMEM((CHUNK,), jnp.int32),
            pltpu.VMEM((CHUNK,), jnp.int32),
            pltpu.VMEM((CHUNK,), jnp.int32),
            pltpu.VMEM((CHUNK,), jnp.int32),
            pltpu.VMEM((SELC,), jnp.int32),
            pltpu.VMEM((GB,), jnp.int32),
            pltpu.VMEM((GB,), jnp.int32),
            pltpu.VMEM((GB, D), jnp.float32),
            pltpu.VMEM((GB,), jnp.int32),
            pltpu.VMEM((GB,), jnp.int32),
            pltpu.VMEM((GB, D), jnp.float32),
            pltpu.SemaphoreType.DMA,
            pltpu.SemaphoreType.DMA,
            pltpu.SemaphoreType.DMA,
        ],
        compiler_params=pltpu.CompilerParams(needs_layout_passes=False),
    )
    def k(fts_hbm, u_hbm, v_hbm, out_hbm, acc, ub0, ub1, vb0, vb1, selc,
          selvA, seldA, rowsA, selvB, seldB, rowsB, esem, gsemA, gsemB):
        wid = lax.axis_index("s") * 2 + lax.axis_index("c")
        lo = wid * RPW

        zero16 = jnp.zeros((16,), jnp.float32)

        def zbody(i, _):
            for j in range(D // 16):
                acc[i, pl.ds(j * 16, 16)] = zero16
            return 0

        lax.fori_loop(0, RPW + 1, zbody, 0)

        def start_chunk(c, ubuf, vbuf):
            pltpu.async_copy(u_hbm.at[pl.ds(c * CHUNK, CHUNK)], ubuf, esem)
            pltpu.async_copy(v_hbm.at[pl.ds(c * CHUNK, CHUNK)], vbuf, esem)

        def wait_chunk(c, ubuf, vbuf):
            pltpu.make_async_copy(u_hbm.at[pl.ds(c * CHUNK, CHUNK)], ubuf, esem).wait()
            pltpu.make_async_copy(v_hbm.at[pl.ds(c * CHUNK, CHUNK)], vbuf, esem).wait()

        def decode_issue(selv, seld, rows, gsem):
            def dbody(j, _):
                sl = pl.ds(j * 16, 16)
                comb = selc[sl]
                selv[sl] = comb >> DSTBITS
                seld[sl] = comb & ((1 << DSTBITS) - 1)
                return 0

            lax.fori_loop(0, GB // 16, dbody, 0)
            pltpu.async_copy(fts_hbm.at[selv], rows, gsem)

        def wait_accum(selv, seld, rows, gsem):
            pltpu.make_async_copy(fts_hbm.at[selv], rows, gsem).wait()

            def abody(g, _):
                dstv = seld[pl.ds(g * 16, 16)]
                dsts = [dstv[t] for t in range(16)]
                for t in range(16):
                    i = g * 16 + t
                    for j in range(D // 16):
                        sl = pl.ds(j * 16, 16)
                        acc[dsts[t], sl] = jnp.maximum(acc[dsts[t], sl], rows[i, sl])
                return 0

            lax.fori_loop(0, GB // 16, abody, 0)

        def fire_cond(state):
            cm1_vec, par, pend = state
            return cm1_vec[0] >= GB - 1

        def fire_body(state):
            cm1_vec, par, pend = state

            @pl.when(par == 0)
            def _():
                decode_issue(selvA, seldA, rowsA, gsemA)

                @pl.when(pend == 1)
                def _():
                    wait_accum(selvB, seldB, rowsB, gsemB)

            @pl.when(par == 1)
            def _():
                decode_issue(selvB, seldB, rowsB, gsemB)

                @pl.when(pend == 1)
                def _():
                    wait_accum(selvA, seldA, rowsA, gsemA)

            for i in range(SLACK // 16):
                selc[pl.ds(i * 16, 16)] = selc[pl.ds(GB + i * 16, 16)]

            return (cm1_vec - GB, 1 - par, jnp.int32(1))

        ones16 = jnp.ones((16,), jnp.int32)

        def scan_chunk(ubuf, vbuf, state):
            def group_body(g, state):
                cm1_vec, par, pend = state
                base = g * (GROUP * 16)
                sls = [pl.ds(base + t * 16, 16) for t in range(GROUP)]
                uvs = [ubuf[sl] for sl in sls]
                vvs = [vbuf[sl] for sl in sls]
                rels = [uv - lo for uv in uvs]
                msks = [(rel >= 0) & (rel < RPW) for rel in rels]
                poss = [plsc.cumsum(ones16, mask=msk) for msk in msks]
                npcs = [plsc.all_reduce_population_count(msk) for msk in msks]
                combs = [(vv << DSTBITS) | rel for vv, rel in zip(vvs, rels)]
                cm1s = [cm1_vec]
                for t in range(GROUP - 1):
                    cm1s.append(cm1s[-1] + npcs[t])
                for t in range(GROUP):
                    plsc.store_scatter(selc, [cm1s[t] + poss[t]], combs[t],
                                       mask=msks[t])
                cm1_vec = cm1s[-1] + npcs[GROUP - 1]
                return lax.while_loop(fire_cond, fire_body, (cm1_vec, par, pend))

            return lax.fori_loop(0, NGROUPS, group_body, state)

        start_chunk(0, ub0, vb0)
        state = (jnp.full((16,), -1, jnp.int32), jnp.int32(0), jnp.int32(0))

        def chunk_body(c2, state):
            a = 2 * c2
            start_chunk(a + 1, ub1, vb1)
            wait_chunk(a, ub0, vb0)
            state = scan_chunk(ub0, vb0, state)

            @pl.when(a + 2 < NCHUNKS)
            def _():
                start_chunk(a + 2, ub0, vb0)

            wait_chunk(a + 1, ub1, vb1)
            state = scan_chunk(ub1, vb1, state)
            return state

        cm1_vec, par, pend = lax.fori_loop(0, NCHUNKS // 2, chunk_body, state)

        lane = lax.iota(jnp.int32, 16)

        def pbody(i, _):
            sl = pl.ds(i * 16, 16)
            live = lane + (i * 16) <= cm1_vec
            selc[sl] = jnp.where(live, selc[sl], RPW)
            return 0

        lax.fori_loop(0, SELC // 16, pbody, 0)

        tail_cnt = jnp.where(cm1_vec[0] >= 0,
                             jnp.full((16,), GB - 1, jnp.int32),
                             jnp.full((16,), -1, jnp.int32))
        cm1_vec, par, pend = lax.while_loop(
            fire_cond, fire_body, (tail_cnt, par, pend))

        @pl.when((pend == 1) & (par == 1))
        def _():
            wait_accum(selvA, seldA, rowsA, gsemA)

        @pl.when((pend == 1) & (par == 0))
        def _():
            wait_accum(selvB, seldB, rowsB, gsemB)

        pltpu.sync_copy(acc.at[pl.ds(0, RPW)], out_hbm.at[pl.ds(lo, RPW)])

    return k(fts, u_arr, v_arr)


def _tc_body(fts_ref, agg_ref, w1_ref, w2_ref, out_ref):
    y = jnp.dot(fts_ref[...], w1_ref[...], preferred_element_type=jnp.float32)
    y = y + jnp.dot(agg_ref[...], w2_ref[...], preferred_element_type=jnp.float32)
    nrm = jnp.sum(y * y, axis=1, keepdims=True)
    out_ref[...] = y * lax.rsqrt(nrm)


def _tc_linear(fts, agg, W_l):
    w1 = W_l[:, :D].T
    w2 = W_l[:, D:].T
    B = 1000
    grid = (N // B,)
    return pl.pallas_call(
        _tc_body,
        grid=grid,
        in_specs=[
            pl.BlockSpec((B, D), lambda i: (i, 0)),
            pl.BlockSpec((B, D), lambda i: (i, 0)),
            pl.BlockSpec((D, D), lambda i: (0, 0)),
            pl.BlockSpec((D, D), lambda i: (0, 0)),
        ],
        out_specs=pl.BlockSpec((B, D), lambda i: (i, 0)),
        out_shape=jax.ShapeDtypeStruct((N, D), jnp.float32),
    )(fts, agg, w1, w2)


def kernel(fts, edge_index, W_l):
    agg = _sc_agg(fts, edge_index[0], edge_index[1])[:N]
    return _tc_linear(fts, agg, W_l)


# speedup vs baseline: 2.1768x; 1.5600x over previous
import functools
import jax
import jax.numpy as jnp
from jax import lax
from jax.experimental import pallas as pl
from jax.experimental.pallas import tpu as pltpu
from jax.experimental.pallas import tpu_sc as plsc

N = 10000
E = 320000
D = 128

NW = 32
RPW = 320
NPAD = NW * RPW
CHUNK = 8000
NCHUNKS = E // CHUNK
VECS = CHUNK // 16
GROUP = 10
NGROUPS = VECS // GROUP
GB = 128
SLACK = GROUP * 16
SELC = GB + SLACK
DSTBITS = 9


def _sc_agg(fts, u_arr, v_arr):
    mesh = plsc.VectorSubcoreMesh(core_axis_name="c", subcore_axis_name="s")

    @functools.partial(
        pl.kernel,
        mesh=mesh,
        out_type=jax.ShapeDtypeStruct((NPAD, D), jnp.float32),
        scratch_types=[
            pltpu.VMEM((RPW + 1, D), jnp.float32),
            pltpu.VMEM((CHUNK,), jnp.int32),
            pltpu.VMEM((CHUNK,), jnp.int32),
            pltpu.VMEM((CHUNK,), jnp.int32),
            pltpu.VMEM((CHUNK,), jnp.int32),
            pltpu.VMEM((SELC,), jnp.int32),
            pltpu.VMEM((GB,), jnp.int32),
            pltpu.VMEM((GB,), jnp.int32),
            pltpu.VMEM((GB, D), jnp.float32),
            pltpu.VMEM((GB,), jnp.int32),
            pltpu.VMEM((GB,), jnp.int32),
            pltpu.VMEM((GB, D), jnp.float32),
            pltpu.SemaphoreType.DMA,
            pltpu.SemaphoreType.DMA,
            pltpu.SemaphoreType.DMA,
        ],
        compiler_params=pltpu.CompilerParams(needs_layout_passes=False),
    )
    def k(fts_hbm, u_hbm, v_hbm, out_hbm, acc, ub0, ub1, vb0, vb1, selc,
          selvA, seldA, rowsA, selvB, seldB, rowsB, esem, gsemA, gsemB):
        wid = lax.axis_index("s") * 2 + lax.axis_index("c")
        lo = wid * RPW

        zero16 = jnp.zeros((16,), jnp.float32)

        def zbody(i, _):
            for j in range(D // 16):
                acc[i, pl.ds(j * 16, 16)] = zero16
            return 0

        lax.fori_loop(0, RPW + 1, zbody, 0)

        def start_chunk(c, ubuf, vbuf):
            pltpu.async_copy(u_hbm.at[pl.ds(c * CHUNK, CHUNK)], ubuf, esem)
            pltpu.async_copy(v_hbm.at[pl.ds(c * CHUNK, CHUNK)], vbuf, esem)

        def wait_chunk(c, ubuf, vbuf):
            pltpu.make_async_copy(u_hbm.at[pl.ds(c * CHUNK, CHUNK)], ubuf, esem).wait()
            pltpu.make_async_copy(v_hbm.at[pl.ds(c * CHUNK, CHUNK)], vbuf, esem).wait()

        def decode_issue(selv, seld, rows, gsem):
            def dbody(j, _):
                sl = pl.ds(j * 16, 16)
                comb = selc[sl]
                selv[sl] = comb >> DSTBITS
                seld[sl] = comb & ((1 << DSTBITS) - 1)
                return 0

            lax.fori_loop(0, GB // 16, dbody, 0)
            pltpu.async_copy(fts_hbm.at[selv], rows, gsem)

        def wait_accum(selv, seld, rows, gsem):
            pltpu.make_async_copy(fts_hbm.at[selv], rows, gsem).wait()

            def abody(g, _):
                dstv = seld[pl.ds(g * 16, 16)]
                dsts = [dstv[t] for t in range(16)]
                sls = [pl.ds(j * 16, 16) for j in range(D // 16)]
                for t in range(16):
                    i = g * 16 + t
                    d = dsts[t]
                    avs = [acc[d, sl] for sl in sls]
                    rvs = [rows[i, sl] for sl in sls]
                    mvs = [jnp.maximum(a, r) for a, r in zip(avs, rvs)]
                    for j in range(D // 16):
                        acc[d, sls[j]] = mvs[j]
                return 0

            lax.fori_loop(0, GB // 16, abody, 0)

        def fire_cond(state):
            cm1_vec, par, pend = state
            return cm1_vec[0] >= GB - 1

        def fire_body(state):
            cm1_vec, par, pend = state

            @pl.when(par == 0)
            def _():
                decode_issue(selvA, seldA, rowsA, gsemA)

                @pl.when(pend == 1)
                def _():
                    wait_accum(selvB, seldB, rowsB, gsemB)

            @pl.when(par == 1)
            def _():
                decode_issue(selvB, seldB, rowsB, gsemB)

                @pl.when(pend == 1)
                def _():
                    wait_accum(selvA, seldA, rowsA, gsemA)

            for i in range(SLACK // 16):
                selc[pl.ds(i * 16, 16)] = selc[pl.ds(GB + i * 16, 16)]

            return (cm1_vec - GB, 1 - par, jnp.int32(1))

        ones16 = jnp.ones((16,), jnp.int32)

        def scan_chunk(ubuf, vbuf, state):
            def group_body(g, state):
                cm1_vec, par, pend = state
                base = g * (GROUP * 16)
                sls = [pl.ds(base + t * 16, 16) for t in range(GROUP)]
                uvs = [ubuf[sl] for sl in sls]
                vvs = [vbuf[sl] for sl in sls]
                rels = [uv - lo for uv in uvs]
                msks = [(rel >= 0) & (rel < RPW) for rel in rels]
                poss = [plsc.cumsum(ones16, mask=msk) for msk in msks]
                npcs = [plsc.all_reduce_population_count(msk) for msk in msks]
                combs = [(vv << DSTBITS) | rel for vv, rel in zip(vvs, rels)]
                cm1s = [cm1_vec]
                for t in range(GROUP - 1):
                    cm1s.append(cm1s[-1] + npcs[t])
                for t in range(GROUP):
                    plsc.store_scatter(selc, [cm1s[t] + poss[t]], combs[t],
                                       mask=msks[t])
                cm1_vec = cm1s[-1] + npcs[GROUP - 1]
                return lax.while_loop(fire_cond, fire_body, (cm1_vec, par, pend))

            return lax.fori_loop(0, NGROUPS, group_body, state)

        start_chunk(0, ub0, vb0)
        state = (jnp.full((16,), -1, jnp.int32), jnp.int32(0), jnp.int32(0))

        def chunk_body(c2, state):
            a = 2 * c2
            start_chunk(a + 1, ub1, vb1)
            wait_chunk(a, ub0, vb0)
            state = scan_chunk(ub0, vb0, state)

            @pl.when(a + 2 < NCHUNKS)
            def _():
                start_chunk(a + 2, ub0, vb0)

            wait_chunk(a + 1, ub1, vb1)
            state = scan_chunk(ub1, vb1, state)
            return state

        cm1_vec, par, pend = lax.fori_loop(0, NCHUNKS // 2, chunk_body, state)

        lane = lax.iota(jnp.int32, 16)

        def pbody(i, _):
            sl = pl.ds(i * 16, 16)
            live = lane + (i * 16) <= cm1_vec
            selc[sl] = jnp.where(live, selc[sl], RPW)
            return 0

        lax.fori_loop(0, SELC // 16, pbody, 0)

        tail_cnt = jnp.where(cm1_vec[0] >= 0,
                             jnp.full((16,), GB - 1, jnp.int32),
                             jnp.full((16,), -1, jnp.int32))
        cm1_vec, par, pend = lax.while_loop(
            fire_cond, fire_body, (tail_cnt, par, pend))

        @pl.when((pend == 1) & (par == 1))
        def _():
            wait_accum(selvA, seldA, rowsA, gsemA)

        @pl.when((pend == 1) & (par == 0))
        def _():
            wait_accum(selvB, seldB, rowsB, gsemB)

        pltpu.sync_copy(acc.at[pl.ds(0, RPW)], out_hbm.at[pl.ds(lo, RPW)])

    return k(fts, u_arr, v_arr)


def _tc_body(fts_ref, agg_ref, w1_ref, w2_ref, out_ref):
    y = jnp.dot(fts_ref[...], w1_ref[...], preferred_element_type=jnp.float32)
    y = y + jnp.dot(agg_ref[...], w2_ref[...], preferred_element_type=jnp.float32)
    nrm = jnp.sum(y * y, axis=1, keepdims=True)
    out_ref[...] = y * lax.rsqrt(nrm)


def _tc_linear(fts, agg, W_l):
    w1 = W_l[:, :D].T
    w2 = W_l[:, D:].T
    B = 1000
    grid = (N // B,)
    return pl.pallas_call(
        _tc_body,
        grid=grid,
        in_specs=[
            pl.BlockSpec((B, D), lambda i: (i, 0)),
            pl.BlockSpec((B, D), lambda i: (i, 0)),
            pl.BlockSpec((D, D), lambda i: (0, 0)),
            pl.BlockSpec((D, D), lambda i: (0, 0)),
        ],
        out_specs=pl.BlockSpec((B, D), lambda i: (i, 0)),
        out_shape=jax.ShapeDtypeStruct((N, D), jnp.float32),
    )(fts, agg, w1, w2)


def kernel(fts, edge_index, W_l):
    agg = _sc_agg(fts, edge_index[0], edge_index[1])[:N]
    return _tc_linear(fts, agg, W_l)


# speedup vs baseline: 2.2671x; 1.0415x over previous
import functools
import jax
import jax.numpy as jnp
from jax import lax
from jax.experimental import pallas as pl
from jax.experimental.pallas import tpu as pltpu
from jax.experimental.pallas import tpu_sc as plsc

N = 10000
E = 320000
D = 128

NW = 32
RPW = 320
NPAD = NW * RPW
CHUNK = 8000
NCHUNKS = E // CHUNK
VECS = CHUNK // 16
GROUP = 10
NGROUPS = VECS // GROUP
GB = 128
SLACK = GROUP * 16
SELC = GB + SLACK
DSTBITS = 9


def _sc_agg(fts, edge_flat):
    mesh = plsc.VectorSubcoreMesh(core_axis_name="c", subcore_axis_name="s")

    @functools.partial(
        pl.kernel,
        mesh=mesh,
        out_type=jax.ShapeDtypeStruct((NPAD, D), jnp.float32),
        scratch_types=[
            pltpu.VMEM((RPW + 1, D), jnp.float32),
            pltpu.VMEM((CHUNK,), jnp.int32),
            pltpu.VMEM((CHUNK,), jnp.int32),
            pltpu.VMEM((CHUNK,), jnp.int32),
            pltpu.VMEM((CHUNK,), jnp.int32),
            pltpu.VMEM((SELC,), jnp.int32),
            pltpu.VMEM((GB,), jnp.int32),
            pltpu.VMEM((GB,), jnp.int32),
            pltpu.VMEM((GB, D), jnp.float32),
            pltpu.VMEM((GB,), jnp.int32),
            pltpu.VMEM((GB,), jnp.int32),
            pltpu.VMEM((GB, D), jnp.float32),
            pltpu.SemaphoreType.DMA,
            pltpu.SemaphoreType.DMA,
            pltpu.SemaphoreType.DMA,
        ],
        compiler_params=pltpu.CompilerParams(needs_layout_passes=False),
    )
    def k(fts_hbm, ei_hbm, out_hbm, acc, ub0, ub1, vb0, vb1, selc,
          selvA, seldA, rowsA, selvB, seldB, rowsB, esem, gsemA, gsemB):
        wid = lax.axis_index("s") * 2 + lax.axis_index("c")
        lo = wid * RPW

        zero16 = jnp.zeros((16,), jnp.float32)

        def zbody(i, _):
            for j in range(D // 16):
                acc[i, pl.ds(j * 16, 16)] = zero16
            return 0

        lax.fori_loop(0, RPW + 1, zbody, 0)

        def start_chunk(c, ubuf, vbuf):
            pltpu.async_copy(ei_hbm.at[pl.ds(c * CHUNK, CHUNK)], ubuf, esem)
            pltpu.async_copy(ei_hbm.at[pl.ds(E + c * CHUNK, CHUNK)], vbuf, esem)

        def wait_chunk(c, ubuf, vbuf):
            pltpu.make_async_copy(ei_hbm.at[pl.ds(c * CHUNK, CHUNK)], ubuf, esem).wait()
            pltpu.make_async_copy(ei_hbm.at[pl.ds(E + c * CHUNK, CHUNK)], vbuf, esem).wait()

        def decode_issue(selv, seld, rows, gsem):
            def dbody(j, _):
                sl = pl.ds(j * 16, 16)
                comb = selc[sl]
                selv[sl] = comb >> DSTBITS
                seld[sl] = comb & ((1 << DSTBITS) - 1)
                return 0

            lax.fori_loop(0, GB // 16, dbody, 0)
            pltpu.async_copy(fts_hbm.at[selv], rows, gsem)

        def wait_accum(selv, seld, rows, gsem):
            pltpu.make_async_copy(fts_hbm.at[selv], rows, gsem).wait()

            def abody(g, _):
                dstv = seld[pl.ds(g * 16, 16)]
                dsts = [dstv[t] for t in range(16)]
                sls = [pl.ds(j * 16, 16) for j in range(D // 16)]
                for t in range(16):
                    i = g * 16 + t
                    d = dsts[t]
                    avs = [acc[d, sl] for sl in sls]
                    rvs = [rows[i, sl] for sl in sls]
                    mvs = [jnp.maximum(a, r) for a, r in zip(avs, rvs)]
                    for j in range(D // 16):
                        acc[d, sls[j]] = mvs[j]
                return 0

            lax.fori_loop(0, GB // 16, abody, 0)

        def fire_cond(state):
            cm1_vec, par, pend = state
            return cm1_vec[0] >= GB - 1

        def fire_body(state):
            cm1_vec, par, pend = state

            @pl.when(par == 0)
            def _():
                decode_issue(selvA, seldA, rowsA, gsemA)

                @pl.when(pend == 1)
                def _():
                    wait_accum(selvB, seldB, rowsB, gsemB)

            @pl.when(par == 1)
            def _():
                decode_issue(selvB, seldB, rowsB, gsemB)

                @pl.when(pend == 1)
                def _():
                    wait_accum(selvA, seldA, rowsA, gsemA)

            for i in range(SLACK // 16):
                selc[pl.ds(i * 16, 16)] = selc[pl.ds(GB + i * 16, 16)]

            return (cm1_vec - GB, 1 - par, jnp.int32(1))

        ones16 = jnp.ones((16,), jnp.int32)

        def scan_chunk(ubuf, vbuf, state):
            def group_body(g, state):
                cm1_vec, par, pend = state
                base = g * (GROUP * 16)
                sls = [pl.ds(base + t * 16, 16) for t in range(GROUP)]
                uvs = [ubuf[sl] for sl in sls]
                vvs = [vbuf[sl] for sl in sls]
                rels = [uv - lo for uv in uvs]
                msks = [(rel >= 0) & (rel < RPW) for rel in rels]
                poss = [plsc.cumsum(ones16, mask=msk) for msk in msks]
                npcs = [plsc.all_reduce_population_count(msk) for msk in msks]
                combs = [(vv << DSTBITS) | rel for vv, rel in zip(vvs, rels)]
                cm1s = [cm1_vec]
                for t in range(GROUP - 1):
                    cm1s.append(cm1s[-1] + npcs[t])
                for t in range(GROUP):
                    plsc.store_scatter(selc, [cm1s[t] + poss[t]], combs[t],
                                       mask=msks[t])
                cm1_vec = cm1s[-1] + npcs[GROUP - 1]
                return lax.while_loop(fire_cond, fire_body, (cm1_vec, par, pend))

            return lax.fori_loop(0, NGROUPS, group_body, state)

        start_chunk(0, ub0, vb0)
        state = (jnp.full((16,), -1, jnp.int32), jnp.int32(0), jnp.int32(0))

        def chunk_body(c2, state):
            a = 2 * c2
            start_chunk(a + 1, ub1, vb1)
            wait_chunk(a, ub0, vb0)
            state = scan_chunk(ub0, vb0, state)

            @pl.when(a + 2 < NCHUNKS)
            def _():
                start_chunk(a + 2, ub0, vb0)

            wait_chunk(a + 1, ub1, vb1)
            state = scan_chunk(ub1, vb1, state)
            return state

        cm1_vec, par, pend = lax.fori_loop(0, NCHUNKS // 2, chunk_body, state)

        lane = lax.iota(jnp.int32, 16)

        def pbody(i, _):
            sl = pl.ds(i * 16, 16)
            live = lane + (i * 16) <= cm1_vec
            selc[sl] = jnp.where(live, selc[sl], RPW)
            return 0

        lax.fori_loop(0, SELC // 16, pbody, 0)

        tail_cnt = jnp.where(cm1_vec[0] >= 0,
                             jnp.full((16,), GB - 1, jnp.int32),
                             jnp.full((16,), -1, jnp.int32))
        cm1_vec, par, pend = lax.while_loop(
            fire_cond, fire_body, (tail_cnt, par, pend))

        @pl.when((pend == 1) & (par == 1))
        def _():
            wait_accum(selvA, seldA, rowsA, gsemA)

        @pl.when((pend == 1) & (par == 0))
        def _():
            wait_accum(selvB, seldB, rowsB, gsemB)

        pltpu.sync_copy(acc.at[pl.ds(0, RPW)], out_hbm.at[pl.ds(lo, RPW)])

    return k(fts, edge_flat)


def _tc_body(fts_ref, agg_ref, w_ref, out_ref):
    w = w_ref[...]
    dn = (((1,), (1,)), ((), ()))
    y = lax.dot_general(fts_ref[...], w[:, :D], dn,
                        preferred_element_type=jnp.float32)
    y = y + lax.dot_general(agg_ref[...], w[:, D:], dn,
                            preferred_element_type=jnp.float32)
    nrm = jnp.sum(y * y, axis=1, keepdims=True)
    out_ref[...] = y * lax.rsqrt(nrm)


def _tc_linear(fts, agg_padded, W_l):
    B = 1000
    grid = (N // B,)
    return pl.pallas_call(
        _tc_body,
        grid=grid,
        in_specs=[
            pl.BlockSpec((B, D), lambda i: (i, 0)),
            pl.BlockSpec((B, D), lambda i: (i, 0)),
            pl.BlockSpec((D, 2 * D), lambda i: (0, 0)),
        ],
        out_specs=pl.BlockSpec((B, D), lambda i: (i, 0)),
        out_shape=jax.ShapeDtypeStruct((N, D), jnp.float32),
    )(fts, agg_padded, W_l)


def kernel(fts, edge_index, W_l):
    agg_padded = _sc_agg(fts, edge_index.reshape(-1))
    return _tc_linear(fts, agg_padded, W_l)
